# Initial kernel scaffold; baseline (speedup 1.0000x reference)
import jax
import jax.numpy as jnp
from jax.experimental import pallas as pl


def kernel(x, edge_index, edge_attr, batch, W1, b1, W2, b2, W3, b3, Wp1, bp1, Wp2, bp2):
    raise NotImplementedError("write your pallas kernel here")


# speedup vs baseline: 1.3626x; 1.3626x over previous
import functools

import jax
import jax.numpy as jnp
from jax.experimental import pallas as pl
from jax.experimental.pallas import tpu as pltpu

N = 50000
E = 800000
G = 1024
H = 64
BM = 2000


def _mm_kernel(x_ref, w_ref, o_ref):
    o_ref[...] = jnp.dot(x_ref[...], w_ref[...],
                         preferred_element_type=jnp.float32)


def _matmul(x, w):
    m, k = x.shape
    f = w.shape[1]
    return pl.pallas_call(
        _mm_kernel,
        grid=(m // BM,),
        in_specs=[pl.BlockSpec((BM, k), lambda i: (i, 0)),
                  pl.BlockSpec((k, f), lambda i: (0, 0))],
        out_specs=pl.BlockSpec((BM, f), lambda i: (i, 0)),
        out_shape=jax.ShapeDtypeStruct((m, f), jnp.float32),
    )(x, w)


def _combine_next_kernel(agg_ref, t_ref, d2_ref, b_ref, w_ref, t2_ref):
    h = jnp.maximum(agg_ref[...] + t_ref[...] * d2_ref[...]
                    + b_ref[...][None, :], 0.0)
    t2_ref[...] = jnp.dot(h, w_ref[...], preferred_element_type=jnp.float32)


def _combine_next(agg, t, d2, b, w):
    return pl.pallas_call(
        _combine_next_kernel,
        grid=(N // BM,),
        in_specs=[pl.BlockSpec((BM, H), lambda i: (i, 0)),
                  pl.BlockSpec((BM, H), lambda i: (i, 0)),
                  pl.BlockSpec((BM, 1), lambda i: (i, 0)),
                  pl.BlockSpec((H,), lambda i: (0,)),
                  pl.BlockSpec((H, H), lambda i: (0, 0))],
        out_specs=pl.BlockSpec((BM, H), lambda i: (i, 0)),
        out_shape=jax.ShapeDtypeStruct((N, H), jnp.float32),
    )(agg, t, d2, b, w)


def _combine_kernel(agg_ref, t_ref, d2_ref, b_ref, h_ref):
    h_ref[...] = jnp.maximum(agg_ref[...] + t_ref[...] * d2_ref[...]
                             + b_ref[...][None, :], 0.0)


def _combine(agg, t, d2, b):
    return pl.pallas_call(
        _combine_kernel,
        grid=(N // BM,),
        in_specs=[pl.BlockSpec((BM, H), lambda i: (i, 0)),
                  pl.BlockSpec((BM, H), lambda i: (i, 0)),
                  pl.BlockSpec((BM, 1), lambda i: (i, 0)),
                  pl.BlockSpec((H,), lambda i: (0,))],
        out_specs=pl.BlockSpec((BM, H), lambda i: (i, 0)),
        out_shape=jax.ShapeDtypeStruct((N, H), jnp.float32),
    )(agg, t, d2, b)


def _meanpool_kernel(h_ref, batch_ref, sum_ref, cnt_ref):
    i = pl.program_id(0)

    @pl.when(i == 0)
    def _init():
        sum_ref[...] = jnp.zeros_like(sum_ref)
        cnt_ref[...] = jnp.zeros_like(cnt_ref)

    b = batch_ref[...]
    onehot = (b == jax.lax.broadcasted_iota(jnp.int32, (1, G), 1)
              ).astype(jnp.float32)
    sum_ref[...] += jnp.dot(onehot.T, h_ref[...],
                            preferred_element_type=jnp.float32)
    cnt_ref[...] += jnp.sum(onehot, axis=0)


def _meanpool(h, batch):
    return pl.pallas_call(
        _meanpool_kernel,
        grid=(N // BM,),
        in_specs=[pl.BlockSpec((BM, H), lambda i: (i, 0)),
                  pl.BlockSpec((BM, 1), lambda i: (i, 0))],
        out_specs=[pl.BlockSpec((G, H), lambda i: (0, 0)),
                   pl.BlockSpec((G,), lambda i: (0,))],
        out_shape=[jax.ShapeDtypeStruct((G, H), jnp.float32),
                   jax.ShapeDtypeStruct((G,), jnp.float32)],
    )(h, batch)


def _head_kernel(sum_ref, cnt_ref, mx_ref, wp1_ref, bp1_ref, wp2_ref,
                 bp2_ref, o_ref):
    cnt = jnp.maximum(cnt_ref[...], 1.0)
    mean = sum_ref[...] / cnt[:, None]
    mx = mx_ref[...]
    mx = jnp.where(jnp.isfinite(mx), mx, 0.0)
    g = jnp.concatenate([mean, mx], axis=1)
    z = jnp.maximum(jnp.dot(g, wp1_ref[...],
                            preferred_element_type=jnp.float32)
                    + bp1_ref[...][None, :], 0.0)
    o_ref[...] = (jnp.dot(z, wp2_ref[...], preferred_element_type=jnp.float32)
                  + bp2_ref[...][None, :])


def _head(sums, cnt, mx, wp1, bp1, wp2, bp2):
    return pl.pallas_call(
        _head_kernel,
        out_shape=jax.ShapeDtypeStruct((G, 1), jnp.float32),
    )(sums, cnt, mx, wp1, bp1, wp2, bp2)


def kernel(x, edge_index, edge_attr, batch, W1, b1, W2, b2, W3, b3,
           Wp1, bp1, Wp2, bp2):
    del edge_attr
    src = edge_index[0].astype(jnp.int32)
    dst = edge_index[1].astype(jnp.int32)
    batch32 = batch.astype(jnp.int32)

    deg = jnp.zeros((N,), jnp.float32).at[dst].add(1.0) + 1.0
    dinv = deg ** -0.5
    d2 = (dinv * dinv)[:, None]
    norm = dinv[src] * dinv[dst]

    def edge_pass(t):
        msg = t[src] * norm[:, None]
        return jnp.zeros((N, H), jnp.float32).at[dst].add(msg)

    xp = jnp.pad(x, ((0, 0), (0, 16 - x.shape[1])))
    W1p = jnp.pad(W1, ((0, 16 - W1.shape[0]), (0, 0)))

    t = _matmul(xp, W1p)
    t = _combine_next(edge_pass(t), t, d2, b1, W2)
    t = _combine_next(edge_pass(t), t, d2, b2, W3)
    h3 = _combine(edge_pass(t), t, d2, b3)

    sums, cnt = _meanpool(h3, batch32[:, None])
    mx = jax.ops.segment_max(h3, batch32, num_segments=G)
    return _head(sums, cnt, mx, Wp1, bp1, Wp2, bp2)


# speedup vs baseline: 9.8243x; 7.2101x over previous
import functools

import jax
import jax.numpy as jnp
from jax import lax
from jax.experimental import pallas as pl
from jax.experimental.pallas import tpu as pltpu
from jax.experimental.pallas import tpu_sc as plsc

N = 50000
E = 800000
G = 1024
H = 64
F2 = H // 2
NPAD = 50048
EPAD = 802816
RPT = NPAD // 16
CH = 128
EPT = EPAD // 16
DPT = EPAD // 32
BM = 3128

_mesh = plsc.VectorSubcoreMesh(core_axis_name="c", subcore_axis_name="s")


def _zero_vmem(ref, rows, cols):
    z = jnp.zeros((16,), jnp.float32)

    @pl.loop(0, rows)
    def _(i):
        for j in range(cols // 16):
            ref[i, pl.ds(j * 16, 16)] = z


def _deg_body(dst_hbm, out0_hbm, out1_hbm, acc, idx_v, ones_v, zer_v, sem):
    c = lax.axis_index("c")
    s = lax.axis_index("s")
    for j in range(CH // 16):
        ones_v[pl.ds(j * 16, 16)] = jnp.full((16,), 1.0, jnp.float32)
    for j in range(RPT // 16 + 1):
        zer_v[pl.ds(j * 16, 16)] = jnp.zeros((16,), jnp.float32)
    pltpu.sync_copy(zer_v.at[pl.ds(0, RPT)], acc.at[pl.ds(s * RPT, RPT)])
    plsc.subcore_barrier()

    base = c * (EPAD // 2) + s * DPT

    @pl.loop(0, DPT // CH)
    def _(i):
        pltpu.sync_copy(dst_hbm.at[pl.ds(base + i * CH, CH)], idx_v)
        pltpu.sync_copy(ones_v, acc.at[idx_v], add=True)

    plsc.subcore_barrier()
    pltpu.sync_copy(acc.at[pl.ds(s * RPT, RPT)], zer_v.at[pl.ds(0, RPT)])

    @pl.when(c == 0)
    def _():
        pltpu.sync_copy(zer_v.at[pl.ds(0, RPT)],
                        out0_hbm.at[pl.ds(s * RPT, RPT)])

    @pl.when(c == 1)
    def _():
        pltpu.sync_copy(zer_v.at[pl.ds(0, RPT)],
                        out1_hbm.at[pl.ds(s * RPT, RPT)])


_deg_sc = functools.partial(
    pl.kernel,
    out_type=[jax.ShapeDtypeStruct((NPAD,), jnp.float32),
              jax.ShapeDtypeStruct((NPAD,), jnp.float32)],
    mesh=_mesh,
    scratch_types=[
        pltpu.VMEM_SHARED((NPAD,), jnp.float32),
        pltpu.VMEM((CH,), jnp.int32),
        pltpu.VMEM((CH,), jnp.float32),
        pltpu.VMEM((RPT + 16,), jnp.float32),
        pltpu.SemaphoreType.DMA,
    ],
)(_deg_body)


def _edge_body(src_hbm, dst_hbm, ts0_hbm, ts1_hbm, out0_hbm, out1_hbm,
               acc, src_v, dst_v, rows_v, zer_v, gsem, ssem):
    c = lax.axis_index("c")
    s = lax.axis_index("s")
    _zero_vmem(zer_v, CH, F2)

    rb = s * RPT

    @pl.loop(0, RPT // CH)
    def _(i):
        pltpu.sync_copy(zer_v, acc.at[pl.ds(rb + i * CH, CH)])
    rem = RPT - (RPT // CH) * CH
    pltpu.sync_copy(zer_v.at[pl.ds(0, rem)],
                    acc.at[pl.ds(rb + (RPT // CH) * CH, rem)])
    plsc.subcore_barrier()

    base = s * EPT

    def run(ts_hbm, out_hbm):
        @pl.loop(0, EPT // CH)
        def _(i):
            eb = base + i * CH
            pltpu.sync_copy(src_hbm.at[pl.ds(eb, CH)], src_v)
            pltpu.sync_copy(dst_hbm.at[pl.ds(eb, CH)], dst_v)
            pltpu.async_copy(ts_hbm.at[src_v], rows_v, gsem).wait()
            pltpu.async_copy(rows_v, acc.at[dst_v], ssem, add=True).wait()

        plsc.subcore_barrier()
        pltpu.sync_copy(acc.at[pl.ds(rb, RPT)], out_hbm.at[pl.ds(rb, RPT)])

    @pl.when(c == 0)
    def _():
        run(ts0_hbm, out0_hbm)

    @pl.when(c == 1)
    def _():
        run(ts1_hbm, out1_hbm)


_edge_sc = functools.partial(
    pl.kernel,
    out_type=[jax.ShapeDtypeStruct((NPAD, F2), jnp.float32),
              jax.ShapeDtypeStruct((NPAD, F2), jnp.float32)],
    mesh=_mesh,
    compiler_params=pltpu.CompilerParams(use_tc_tiling_on_sc=False),
    scratch_types=[
        pltpu.VMEM_SHARED((NPAD, F2), jnp.float32),
        pltpu.VMEM((CH,), jnp.int32),
        pltpu.VMEM((CH,), jnp.int32),
        pltpu.VMEM((CH, F2), jnp.float32),
        pltpu.VMEM((CH, F2), jnp.float32),
        pltpu.SemaphoreType.DMA,
        pltpu.SemaphoreType.DMA,
    ],
)(_edge_body)


def _prep_kernel(x_ref, w_ref, dg0_ref, dg1_ref, dinv_ref, ts0_ref, ts1_ref):
    deg = dg0_ref[...] + dg1_ref[...] + 1.0
    dinv = lax.rsqrt(deg)
    dinv_ref[...] = dinv
    ts = jnp.dot(x_ref[...], w_ref[...],
                 preferred_element_type=jnp.float32) * dinv
    ts0_ref[...] = ts[:, :F2]
    ts1_ref[...] = ts[:, F2:]


def _prep(xp, w1p, dg0, dg1):
    return pl.pallas_call(
        _prep_kernel,
        grid=(NPAD // BM,),
        in_specs=[pl.BlockSpec((BM, 16), lambda i: (i, 0)),
                  pl.BlockSpec((16, H), lambda i: (0, 0)),
                  pl.BlockSpec((BM, 1), lambda i: (i, 0)),
                  pl.BlockSpec((BM, 1), lambda i: (i, 0))],
        out_specs=[pl.BlockSpec((BM, 1), lambda i: (i, 0)),
                   pl.BlockSpec((BM, F2), lambda i: (i, 0)),
                   pl.BlockSpec((BM, F2), lambda i: (i, 0))],
        out_shape=[jax.ShapeDtypeStruct((NPAD, 1), jnp.float32),
                   jax.ShapeDtypeStruct((NPAD, F2), jnp.float32),
                   jax.ShapeDtypeStruct((NPAD, F2), jnp.float32)],
    )(xp, w1p, dg0, dg1)


def _combine_next_kernel(s0_ref, s1_ref, t0_ref, t1_ref, dinv_ref, b_ref,
                         w_ref, o0_ref, o1_ref):
    dinv = dinv_ref[...]
    h0 = jnp.maximum(dinv * (s0_ref[...] + t0_ref[...])
                     + b_ref[...][None, :F2], 0.0)
    h1 = jnp.maximum(dinv * (s1_ref[...] + t1_ref[...])
                     + b_ref[...][None, F2:], 0.0)
    h = jnp.concatenate([h0, h1], axis=1)
    ts = jnp.dot(h, w_ref[...], preferred_element_type=jnp.float32) * dinv
    o0_ref[...] = ts[:, :F2]
    o1_ref[...] = ts[:, F2:]


def _combine_next(s0, s1, t0, t1, dinv, b, w):
    return pl.pallas_call(
        _combine_next_kernel,
        grid=(NPAD // BM,),
        in_specs=[pl.BlockSpec((BM, F2), lambda i: (i, 0)),
                  pl.BlockSpec((BM, F2), lambda i: (i, 0)),
                  pl.BlockSpec((BM, F2), lambda i: (i, 0)),
                  pl.BlockSpec((BM, F2), lambda i: (i, 0)),
                  pl.BlockSpec((BM, 1), lambda i: (i, 0)),
                  pl.BlockSpec((H,), lambda i: (0,)),
                  pl.BlockSpec((H, H), lambda i: (0, 0))],
        out_specs=[pl.BlockSpec((BM, F2), lambda i: (i, 0)),
                   pl.BlockSpec((BM, F2), lambda i: (i, 0))],
        out_shape=[jax.ShapeDtypeStruct((NPAD, F2), jnp.float32),
                   jax.ShapeDtypeStruct((NPAD, F2), jnp.float32)],
    )(s0, s1, t0, t1, dinv, b, w)


def _combine_last_kernel(s0_ref, s1_ref, t0_ref, t1_ref, dinv_ref, b_ref,
                         h_ref):
    dinv = dinv_ref[...]
    h0 = jnp.maximum(dinv * (s0_ref[...] + t0_ref[...])
                     + b_ref[...][None, :F2], 0.0)
    h1 = jnp.maximum(dinv * (s1_ref[...] + t1_ref[...])
                     + b_ref[...][None, F2:], 0.0)
    h_ref[...] = jnp.concatenate([h0, h1], axis=1)


def _combine_last(s0, s1, t0, t1, dinv, b):
    return pl.pallas_call(
        _combine_last_kernel,
        grid=(NPAD // BM,),
        in_specs=[pl.BlockSpec((BM, F2), lambda i: (i, 0)),
                  pl.BlockSpec((BM, F2), lambda i: (i, 0)),
                  pl.BlockSpec((BM, F2), lambda i: (i, 0)),
                  pl.BlockSpec((BM, F2), lambda i: (i, 0)),
                  pl.BlockSpec((BM, 1), lambda i: (i, 0)),
                  pl.BlockSpec((H,), lambda i: (0,))],
        out_specs=pl.BlockSpec((BM, H), lambda i: (i, 0)),
        out_shape=jax.ShapeDtypeStruct((NPAD, H), jnp.float32),
    )(s0, s1, t0, t1, dinv, b)


def _meanpool_kernel(h_ref, batch_ref, sum_ref, cnt_ref):
    i = pl.program_id(0)

    @pl.when(i == 0)
    def _init():
        sum_ref[...] = jnp.zeros_like(sum_ref)
        cnt_ref[...] = jnp.zeros_like(cnt_ref)

    b = batch_ref[...]
    onehot = (b == jax.lax.broadcasted_iota(jnp.int32, (1, G), 1)
              ).astype(jnp.float32)
    sum_ref[...] += jnp.dot(onehot.T, h_ref[...],
                            preferred_element_type=jnp.float32)
    cnt_ref[...] += jnp.sum(onehot, axis=0)


def _meanpool(h, batch2d):
    return pl.pallas_call(
        _meanpool_kernel,
        grid=(NPAD // BM,),
        in_specs=[pl.BlockSpec((BM, H), lambda i: (i, 0)),
                  pl.BlockSpec((BM, 1), lambda i: (i, 0))],
        out_specs=[pl.BlockSpec((G, H), lambda i: (0, 0)),
                   pl.BlockSpec((G,), lambda i: (0,))],
        out_shape=[jax.ShapeDtypeStruct((G, H), jnp.float32),
                   jax.ShapeDtypeStruct((G,), jnp.float32)],
    )(h, batch2d)


def _head_kernel(sum_ref, cnt_ref, mx_ref, wp1_ref, bp1_ref, wp2_ref,
                 bp2_ref, o_ref):
    cnt = jnp.maximum(cnt_ref[...], 1.0)
    mean = sum_ref[...] / cnt[:, None]
    mx = mx_ref[...]
    mx = jnp.where(jnp.isfinite(mx), mx, 0.0)
    g = jnp.concatenate([mean, mx], axis=1)
    z = jnp.maximum(jnp.dot(g, wp1_ref[...],
                            preferred_element_type=jnp.float32)
                    + bp1_ref[...][None, :], 0.0)
    o_ref[...] = (jnp.dot(z, wp2_ref[...], preferred_element_type=jnp.float32)
                  + bp2_ref[...][None, :])


def _head(sums, cnt, mx, wp1, bp1, wp2, bp2):
    return pl.pallas_call(
        _head_kernel,
        out_shape=jax.ShapeDtypeStruct((G, 1), jnp.float32),
    )(sums, cnt, mx, wp1, bp1, wp2, bp2)


def kernel(x, edge_index, edge_attr, batch, W1, b1, W2, b2, W3, b3,
           Wp1, bp1, Wp2, bp2):
    del edge_attr
    src = jnp.concatenate([edge_index[0].astype(jnp.int32),
                           jnp.zeros((EPAD - E,), jnp.int32)])
    dst = jnp.concatenate([edge_index[1].astype(jnp.int32),
                           jnp.full((EPAD - E,), N, jnp.int32)])
    batch32 = batch.astype(jnp.int32)
    batch_p = jnp.concatenate([batch32, jnp.full((NPAD - N,), G, jnp.int32)])

    xp = jnp.pad(x, ((0, NPAD - N), (0, 16 - x.shape[1])))
    w1p = jnp.pad(W1, ((0, 16 - W1.shape[0]), (0, 0)))

    dg0, dg1 = _deg_sc(dst)
    dinv, t0, t1 = _prep(xp, w1p, dg0[:, None], dg1[:, None])

    s0, s1 = _edge_sc(src, dst, t0, t1)
    t0, t1 = _combine_next(s0, s1, t0, t1, dinv, b1, W2)
    s0, s1 = _edge_sc(src, dst, t0, t1)
    t0, t1 = _combine_next(s0, s1, t0, t1, dinv, b2, W3)
    s0, s1 = _edge_sc(src, dst, t0, t1)
    h3 = _combine_last(s0, s1, t0, t1, dinv, b3)

    sums, cnt = _meanpool(h3, batch_p[:, None])
    mx = jax.ops.segment_max(h3[:N], batch32, num_segments=G)
    return _head(sums, cnt, mx, Wp1, bp1, Wp2, bp2)


# speedup vs baseline: 22.5819x; 2.2986x over previous
import functools

import jax
import jax.numpy as jnp
from jax import lax
from jax.experimental import pallas as pl
from jax.experimental.pallas import tpu as pltpu
from jax.experimental.pallas import tpu_sc as plsc

N = 50000
E = 800000
G = 1024
H = 64
F2 = H // 2
NPAD = 50048
EPAD = 802816
RPT = NPAD // 16
CH = 128
EPT = EPAD // 16
DPT = EPAD // 32
BM = 3128

_mesh = plsc.VectorSubcoreMesh(core_axis_name="c", subcore_axis_name="s")


def _zero_vmem(ref, rows, cols):
    z = jnp.zeros((16,), jnp.float32)

    @pl.loop(0, rows)
    def _(i):
        for j in range(cols // 16):
            ref[i, pl.ds(j * 16, 16)] = z


def _deg_body(dst_hbm, out0_hbm, out1_hbm, acc, idx_v, ones_v, zer_v, sem):
    c = lax.axis_index("c")
    s = lax.axis_index("s")
    for j in range(CH // 16):
        ones_v[pl.ds(j * 16, 16)] = jnp.full((16,), 1.0, jnp.float32)
    for j in range(RPT // 16 + 1):
        zer_v[pl.ds(j * 16, 16)] = jnp.zeros((16,), jnp.float32)
    pltpu.sync_copy(zer_v.at[pl.ds(0, RPT)], acc.at[pl.ds(s * RPT, RPT)])
    plsc.subcore_barrier()

    base = c * (EPAD // 2) + s * DPT

    @pl.loop(0, DPT // CH)
    def _(i):
        pltpu.sync_copy(dst_hbm.at[pl.ds(base + i * CH, CH)], idx_v)
        pltpu.sync_copy(ones_v, acc.at[idx_v], add=True)

    plsc.subcore_barrier()
    pltpu.sync_copy(acc.at[pl.ds(s * RPT, RPT)], zer_v.at[pl.ds(0, RPT)])

    @pl.when(c == 0)
    def _():
        pltpu.sync_copy(zer_v.at[pl.ds(0, RPT)],
                        out0_hbm.at[pl.ds(s * RPT, RPT)])

    @pl.when(c == 1)
    def _():
        pltpu.sync_copy(zer_v.at[pl.ds(0, RPT)],
                        out1_hbm.at[pl.ds(s * RPT, RPT)])


_deg_sc = functools.partial(
    pl.kernel,
    out_type=[jax.ShapeDtypeStruct((NPAD,), jnp.float32),
              jax.ShapeDtypeStruct((NPAD,), jnp.float32)],
    mesh=_mesh,
    scratch_types=[
        pltpu.VMEM_SHARED((NPAD,), jnp.float32),
        pltpu.VMEM((CH,), jnp.int32),
        pltpu.VMEM((CH,), jnp.float32),
        pltpu.VMEM((RPT + 16,), jnp.float32),
        pltpu.SemaphoreType.DMA,
    ],
)(_deg_body)


R = 4
BLK = 8
NBLK = EPT // CH // BLK


def _edge_body(src_hbm, dst_hbm, ts0_hbm, ts1_hbm, out0_hbm, out1_hbm,
               acc, src_blk, dst_blk, rows, zer_v, gsem, ssem, bsem):
    c = lax.axis_index("c")
    s = lax.axis_index("s")
    _zero_vmem(zer_v, CH, F2)

    rb = s * RPT

    @pl.loop(0, RPT // CH)
    def _(i):
        pltpu.sync_copy(zer_v, acc.at[pl.ds(rb + i * CH, CH)])
    rem = RPT - (RPT // CH) * CH
    pltpu.sync_copy(zer_v.at[pl.ds(0, rem)],
                    acc.at[pl.ds(rb + (RPT // CH) * CH, rem)])
    plsc.subcore_barrier()

    row0 = s * (EPT // CH)

    def load_blk(n, slot):
        pltpu.async_copy(src_hbm.at[pl.ds(row0 + n * BLK, BLK)],
                         src_blk.at[slot], bsem.at[slot])
        pltpu.async_copy(dst_hbm.at[pl.ds(row0 + n * BLK, BLK)],
                         dst_blk.at[slot], bsem.at[slot])

    def wait_blk(slot):
        pltpu.make_async_copy(src_hbm.at[pl.ds(0, BLK)],
                              src_blk.at[slot], bsem.at[slot]).wait()
        pltpu.make_async_copy(dst_hbm.at[pl.ds(0, BLK)],
                              dst_blk.at[slot], bsem.at[slot]).wait()

    def run(ts_hbm, out_hbm):
        def gather_start(slot, j, b):
            pltpu.async_copy(ts_hbm.at[src_blk.at[slot, j]], rows.at[b],
                             gsem.at[b])

        def gather_wait(b):
            pltpu.make_async_copy(ts_hbm.at[src_blk.at[0, 0]], rows.at[b],
                                  gsem.at[b]).wait()

        def scatter_start(slot, j, b):
            pltpu.async_copy(rows.at[b], acc.at[dst_blk.at[slot, j]],
                             ssem.at[b], add=True)

        def scatter_wait(b):
            pltpu.make_async_copy(rows.at[b], acc.at[dst_blk.at[0, 0]],
                                  ssem.at[b]).wait()

        load_blk(0, 0)
        load_blk(1, 1)

        @pl.loop(0, NBLK)
        def _(n):
            slot = lax.rem(n, 3)
            wait_blk(slot)
            for j in range(BLK):
                b = j % R
                if j >= R:
                    scatter_wait(b)
                else:
                    @pl.when(n > 0)
                    def _():
                        scatter_wait(b)
                gather_start(slot, j, b)
                if j >= R - 1:
                    b2 = (j - (R - 1)) % R
                    gather_wait(b2)
                    scatter_start(slot, j - (R - 1), b2)
                else:
                    @pl.when(n > 0)
                    def _():
                        slot_p = lax.rem(n + 2, 3)
                        b2 = (j + BLK - (R - 1)) % R
                        gather_wait(b2)
                        scatter_start(slot_p, j + BLK - (R - 1), b2)
                if j == R:
                    @pl.when(n + 2 < NBLK)
                    def _():
                        load_blk(n + 2, lax.rem(n + 2, 3))

        for d in range(R - 1):
            j = BLK - (R - 1) + d
            b2 = j % R
            gather_wait(b2)
            scatter_start((NBLK - 1) % 3, j, b2)
        for b in range(R):
            scatter_wait(b)

        plsc.subcore_barrier()
        pltpu.sync_copy(acc.at[pl.ds(rb, RPT)], out_hbm.at[pl.ds(rb, RPT)])

    @pl.when(c == 0)
    def _():
        run(ts0_hbm, out0_hbm)

    @pl.when(c == 1)
    def _():
        run(ts1_hbm, out1_hbm)


_edge_sc = functools.partial(
    pl.kernel,
    out_type=[jax.ShapeDtypeStruct((NPAD, F2), jnp.float32),
              jax.ShapeDtypeStruct((NPAD, F2), jnp.float32)],
    mesh=_mesh,
    compiler_params=pltpu.CompilerParams(use_tc_tiling_on_sc=False),
    scratch_types=[
        pltpu.VMEM_SHARED((NPAD, F2), jnp.float32),
        pltpu.VMEM((3, BLK, CH), jnp.int32),
        pltpu.VMEM((3, BLK, CH), jnp.int32),
        pltpu.VMEM((R, CH, F2), jnp.float32),
        pltpu.VMEM((CH, F2), jnp.float32),
        pltpu.SemaphoreType.DMA((R,)),
        pltpu.SemaphoreType.DMA((R,)),
        pltpu.SemaphoreType.DMA((3,)),
    ],
)(_edge_body)


def _prep_kernel(x_ref, w_ref, dg0_ref, dg1_ref, dinv_ref, ts0_ref, ts1_ref):
    deg = dg0_ref[...] + dg1_ref[...] + 1.0
    dinv = lax.rsqrt(deg)
    dinv_ref[...] = dinv
    ts = jnp.dot(x_ref[...], w_ref[...],
                 preferred_element_type=jnp.float32) * dinv
    ts0_ref[...] = ts[:, :F2]
    ts1_ref[...] = ts[:, F2:]


def _prep(xp, w1p, dg0, dg1):
    return pl.pallas_call(
        _prep_kernel,
        grid=(NPAD // BM,),
        in_specs=[pl.BlockSpec((BM, 16), lambda i: (i, 0)),
                  pl.BlockSpec((16, H), lambda i: (0, 0)),
                  pl.BlockSpec((BM, 1), lambda i: (i, 0)),
                  pl.BlockSpec((BM, 1), lambda i: (i, 0))],
        out_specs=[pl.BlockSpec((BM, 1), lambda i: (i, 0)),
                   pl.BlockSpec((BM, F2), lambda i: (i, 0)),
                   pl.BlockSpec((BM, F2), lambda i: (i, 0))],
        out_shape=[jax.ShapeDtypeStruct((NPAD, 1), jnp.float32),
                   jax.ShapeDtypeStruct((NPAD, F2), jnp.float32),
                   jax.ShapeDtypeStruct((NPAD, F2), jnp.float32)],
    )(xp, w1p, dg0, dg1)


def _combine_next_kernel(s0_ref, s1_ref, t0_ref, t1_ref, dinv_ref, b_ref,
                         w_ref, o0_ref, o1_ref):
    dinv = dinv_ref[...]
    h0 = jnp.maximum(dinv * (s0_ref[...] + t0_ref[...])
                     + b_ref[...][None, :F2], 0.0)
    h1 = jnp.maximum(dinv * (s1_ref[...] + t1_ref[...])
                     + b_ref[...][None, F2:], 0.0)
    h = jnp.concatenate([h0, h1], axis=1)
    ts = jnp.dot(h, w_ref[...], preferred_element_type=jnp.float32) * dinv
    o0_ref[...] = ts[:, :F2]
    o1_ref[...] = ts[:, F2:]


def _combine_next(s0, s1, t0, t1, dinv, b, w):
    return pl.pallas_call(
        _combine_next_kernel,
        grid=(NPAD // BM,),
        in_specs=[pl.BlockSpec((BM, F2), lambda i: (i, 0)),
                  pl.BlockSpec((BM, F2), lambda i: (i, 0)),
                  pl.BlockSpec((BM, F2), lambda i: (i, 0)),
                  pl.BlockSpec((BM, F2), lambda i: (i, 0)),
                  pl.BlockSpec((BM, 1), lambda i: (i, 0)),
                  pl.BlockSpec((H,), lambda i: (0,)),
                  pl.BlockSpec((H, H), lambda i: (0, 0))],
        out_specs=[pl.BlockSpec((BM, F2), lambda i: (i, 0)),
                   pl.BlockSpec((BM, F2), lambda i: (i, 0))],
        out_shape=[jax.ShapeDtypeStruct((NPAD, F2), jnp.float32),
                   jax.ShapeDtypeStruct((NPAD, F2), jnp.float32)],
    )(s0, s1, t0, t1, dinv, b, w)


def _combine_last_kernel(s0_ref, s1_ref, t0_ref, t1_ref, dinv_ref, b_ref,
                         h_ref):
    dinv = dinv_ref[...]
    h0 = jnp.maximum(dinv * (s0_ref[...] + t0_ref[...])
                     + b_ref[...][None, :F2], 0.0)
    h1 = jnp.maximum(dinv * (s1_ref[...] + t1_ref[...])
                     + b_ref[...][None, F2:], 0.0)
    h_ref[...] = jnp.concatenate([h0, h1], axis=1)


def _combine_last(s0, s1, t0, t1, dinv, b):
    return pl.pallas_call(
        _combine_last_kernel,
        grid=(NPAD // BM,),
        in_specs=[pl.BlockSpec((BM, F2), lambda i: (i, 0)),
                  pl.BlockSpec((BM, F2), lambda i: (i, 0)),
                  pl.BlockSpec((BM, F2), lambda i: (i, 0)),
                  pl.BlockSpec((BM, F2), lambda i: (i, 0)),
                  pl.BlockSpec((BM, 1), lambda i: (i, 0)),
                  pl.BlockSpec((H,), lambda i: (0,))],
        out_specs=pl.BlockSpec((BM, H), lambda i: (i, 0)),
        out_shape=jax.ShapeDtypeStruct((NPAD, H), jnp.float32),
    )(s0, s1, t0, t1, dinv, b)


def _meanpool_kernel(h_ref, batch_ref, sum_ref, cnt_ref):
    i = pl.program_id(0)

    @pl.when(i == 0)
    def _init():
        sum_ref[...] = jnp.zeros_like(sum_ref)
        cnt_ref[...] = jnp.zeros_like(cnt_ref)

    b = batch_ref[...]
    onehot = (b == jax.lax.broadcasted_iota(jnp.int32, (1, G), 1)
              ).astype(jnp.float32)
    sum_ref[...] += jnp.dot(onehot.T, h_ref[...],
                            preferred_element_type=jnp.float32)
    cnt_ref[...] += jnp.sum(onehot, axis=0)


def _meanpool(h, batch2d):
    return pl.pallas_call(
        _meanpool_kernel,
        grid=(NPAD // BM,),
        in_specs=[pl.BlockSpec((BM, H), lambda i: (i, 0)),
                  pl.BlockSpec((BM, 1), lambda i: (i, 0))],
        out_specs=[pl.BlockSpec((G, H), lambda i: (0, 0)),
                   pl.BlockSpec((G,), lambda i: (0,))],
        out_shape=[jax.ShapeDtypeStruct((G, H), jnp.float32),
                   jax.ShapeDtypeStruct((G,), jnp.float32)],
    )(h, batch2d)


def _head_kernel(sum_ref, cnt_ref, mx_ref, wp1_ref, bp1_ref, wp2_ref,
                 bp2_ref, o_ref):
    cnt = jnp.maximum(cnt_ref[...], 1.0)
    mean = sum_ref[...] / cnt[:, None]
    mx = mx_ref[...]
    mx = jnp.where(jnp.isfinite(mx), mx, 0.0)
    g = jnp.concatenate([mean, mx], axis=1)
    z = jnp.maximum(jnp.dot(g, wp1_ref[...],
                            preferred_element_type=jnp.float32)
                    + bp1_ref[...][None, :], 0.0)
    o_ref[...] = (jnp.dot(z, wp2_ref[...], preferred_element_type=jnp.float32)
                  + bp2_ref[...][None, :])


def _head(sums, cnt, mx, wp1, bp1, wp2, bp2):
    return pl.pallas_call(
        _head_kernel,
        out_shape=jax.ShapeDtypeStruct((G, 1), jnp.float32),
    )(sums, cnt, mx, wp1, bp1, wp2, bp2)


def kernel(x, edge_index, edge_attr, batch, W1, b1, W2, b2, W3, b3,
           Wp1, bp1, Wp2, bp2):
    del edge_attr
    src = jnp.concatenate([edge_index[0].astype(jnp.int32),
                           jnp.zeros((EPAD - E,), jnp.int32)])
    dst = jnp.concatenate([edge_index[1].astype(jnp.int32),
                           jnp.full((EPAD - E,), N, jnp.int32)])
    batch32 = batch.astype(jnp.int32)
    batch_p = jnp.concatenate([batch32, jnp.full((NPAD - N,), G, jnp.int32)])

    xp = jnp.pad(x, ((0, NPAD - N), (0, 16 - x.shape[1])))
    w1p = jnp.pad(W1, ((0, 16 - W1.shape[0]), (0, 0)))

    src2d = src.reshape(EPAD // CH, CH)
    dst2d = dst.reshape(EPAD // CH, CH)

    dg0, dg1 = _deg_sc(dst)
    dinv, t0, t1 = _prep(xp, w1p, dg0[:, None], dg1[:, None])

    s0, s1 = _edge_sc(src2d, dst2d, t0, t1)
    t0, t1 = _combine_next(s0, s1, t0, t1, dinv, b1, W2)
    s0, s1 = _edge_sc(src2d, dst2d, t0, t1)
    t0, t1 = _combine_next(s0, s1, t0, t1, dinv, b2, W3)
    s0, s1 = _edge_sc(src2d, dst2d, t0, t1)
    h3 = _combine_last(s0, s1, t0, t1, dinv, b3)

    sums, cnt = _meanpool(h3, batch_p[:, None])
    mx = jax.ops.segment_max(h3[:N], batch32, num_segments=G)
    return _head(sums, cnt, mx, Wp1, bp1, Wp2, bp2)


# speedup vs baseline: 24.1348x; 1.0688x over previous
import functools

import jax
import jax.numpy as jnp
from jax import lax
from jax.experimental import pallas as pl
from jax.experimental.pallas import tpu as pltpu
from jax.experimental.pallas import tpu_sc as plsc

N = 50000
E = 800000
G = 1024
H = 64
F2 = H // 2
NPAD = 50048
EPAD = 802816
RPT = NPAD // 16
CH = 128
EPT = EPAD // 16
DPT = EPAD // 32
BM = 3128

_mesh = plsc.VectorSubcoreMesh(core_axis_name="c", subcore_axis_name="s")


def _zero_vmem(ref, rows, cols):
    z = jnp.zeros((16,), jnp.float32)

    @pl.loop(0, rows)
    def _(i):
        for j in range(cols // 16):
            ref[i, pl.ds(j * 16, 16)] = z


DR = 4
DBLK = 4
DNBLK = DPT // CH // DBLK


def _deg_body(dst_hbm, out0_hbm, out1_hbm, acc, dst_blk, ones_v, zer_v,
              ssem, bsem):
    c = lax.axis_index("c")
    s = lax.axis_index("s")
    for j in range(CH // 16):
        ones_v[pl.ds(j * 16, 16)] = jnp.full((16,), 1.0, jnp.float32)
    for j in range(RPT // 16 + 1):
        zer_v[pl.ds(j * 16, 16)] = jnp.zeros((16,), jnp.float32)
    pltpu.sync_copy(zer_v.at[pl.ds(0, RPT)], acc.at[pl.ds(s * RPT, RPT)])
    plsc.subcore_barrier()

    row0 = (c * (EPAD // 2) + s * DPT) // CH

    def load_blk(n, slot):
        pltpu.async_copy(dst_hbm.at[pl.ds(row0 + n * DBLK, DBLK)],
                         dst_blk.at[slot], bsem.at[slot])

    def wait_blk(slot):
        pltpu.make_async_copy(dst_hbm.at[pl.ds(0, DBLK)],
                              dst_blk.at[slot], bsem.at[slot]).wait()

    def scatter_wait(b):
        pltpu.make_async_copy(ones_v, acc.at[dst_blk.at[0, 0]],
                              ssem.at[b]).wait()

    load_blk(0, 0)
    load_blk(1, 1)

    @pl.loop(0, DNBLK)
    def _(n):
        slot = lax.rem(n, 4)
        wait_blk(slot)
        for j in range(DBLK):
            @pl.when(n > 0)
            def _():
                scatter_wait(j)
            pltpu.async_copy(ones_v, acc.at[dst_blk.at[slot, j]],
                             ssem.at[j], add=True)
            if j == 2:
                @pl.when(n + 2 < DNBLK)
                def _():
                    load_blk(n + 2, lax.rem(n + 2, 4))

    for b in range(DR):
        scatter_wait(b)
    plsc.subcore_barrier()
    pltpu.sync_copy(acc.at[pl.ds(s * RPT, RPT)], zer_v.at[pl.ds(0, RPT)])

    @pl.when(c == 0)
    def _():
        pltpu.sync_copy(zer_v.at[pl.ds(0, RPT)],
                        out0_hbm.at[pl.ds(s * RPT, RPT)])

    @pl.when(c == 1)
    def _():
        pltpu.sync_copy(zer_v.at[pl.ds(0, RPT)],
                        out1_hbm.at[pl.ds(s * RPT, RPT)])


_deg_sc = functools.partial(
    pl.kernel,
    out_type=[jax.ShapeDtypeStruct((NPAD,), jnp.float32),
              jax.ShapeDtypeStruct((NPAD,), jnp.float32)],
    mesh=_mesh,
    compiler_params=pltpu.CompilerParams(use_tc_tiling_on_sc=False),
    scratch_types=[
        pltpu.VMEM_SHARED((NPAD,), jnp.float32),
        pltpu.VMEM((4, DBLK, CH), jnp.int32),
        pltpu.VMEM((CH,), jnp.float32),
        pltpu.VMEM((RPT + 16,), jnp.float32),
        pltpu.SemaphoreType.DMA((DR,)),
        pltpu.SemaphoreType.DMA((4,)),
    ],
)(_deg_body)


R = 4
BLK = 8
NBLK = EPT // CH // BLK
NIB = 3


def _edge_body(src_hbm, dst_hbm, ts0_hbm, ts1_hbm, out0_hbm, out1_hbm,
               acc, src_blk, dst_blk, rows, zer_v, gsem, ssem, bsem):
    c = lax.axis_index("c")
    s = lax.axis_index("s")
    _zero_vmem(zer_v, CH, F2)

    rb = s * RPT

    @pl.loop(0, RPT // CH)
    def _(i):
        pltpu.sync_copy(zer_v, acc.at[pl.ds(rb + i * CH, CH)])
    rem = RPT - (RPT // CH) * CH
    pltpu.sync_copy(zer_v.at[pl.ds(0, rem)],
                    acc.at[pl.ds(rb + (RPT // CH) * CH, rem)])
    plsc.subcore_barrier()

    row0 = s * (EPT // CH)

    def load_blk(n, slot):
        pltpu.async_copy(src_hbm.at[pl.ds(row0 + n * BLK, BLK)],
                         src_blk.at[slot], bsem.at[slot])
        pltpu.async_copy(dst_hbm.at[pl.ds(row0 + n * BLK, BLK)],
                         dst_blk.at[slot], bsem.at[slot])

    def wait_blk(slot):
        pltpu.make_async_copy(src_hbm.at[pl.ds(0, BLK)],
                              src_blk.at[slot], bsem.at[slot]).wait()
        pltpu.make_async_copy(dst_hbm.at[pl.ds(0, BLK)],
                              dst_blk.at[slot], bsem.at[slot]).wait()

    def run(ts_hbm, out_hbm):
        def gather_start(slot, j, b):
            pltpu.async_copy(ts_hbm.at[src_blk.at[slot, j]], rows.at[b],
                             gsem.at[b])

        def gather_wait(b):
            pltpu.make_async_copy(ts_hbm.at[src_blk.at[0, 0]], rows.at[b],
                                  gsem.at[b]).wait()

        def scatter_start(slot, j, b):
            pltpu.async_copy(rows.at[b], acc.at[dst_blk.at[slot, j]],
                             ssem.at[b], add=True)

        def scatter_wait(b):
            pltpu.make_async_copy(rows.at[b], acc.at[dst_blk.at[0, 0]],
                                  ssem.at[b]).wait()

        load_blk(0, 0)
        load_blk(1, 1)

        @pl.loop(0, NBLK)
        def _(n):
            slot = lax.rem(n, NIB)
            wait_blk(slot)
            for j in range(BLK):
                b = j % R
                if j >= R:
                    scatter_wait(b)
                else:
                    @pl.when(n > 0)
                    def _():
                        scatter_wait(b)
                gather_start(slot, j, b)
                if j >= R - 1:
                    b2 = (j - (R - 1)) % R
                    gather_wait(b2)
                    scatter_start(slot, j - (R - 1), b2)
                else:
                    @pl.when(n > 0)
                    def _():
                        slot_p = lax.rem(n + NIB - 1, NIB)
                        b2 = (j + BLK - (R - 1)) % R
                        gather_wait(b2)
                        scatter_start(slot_p, j + BLK - (R - 1), b2)
                if j == 4:
                    @pl.when(n + 2 < NBLK)
                    def _():
                        load_blk(n + 2, lax.rem(n + 2, NIB))

        for d in range(R - 1):
            j = BLK - (R - 1) + d
            b2 = j % R
            gather_wait(b2)
            scatter_start((NBLK - 1) % NIB, j, b2)
        for b in range(R):
            scatter_wait(b)

        plsc.subcore_barrier()
        pltpu.sync_copy(acc.at[pl.ds(rb, RPT)], out_hbm.at[pl.ds(rb, RPT)])

    @pl.when(c == 0)
    def _():
        run(ts0_hbm, out0_hbm)

    @pl.when(c == 1)
    def _():
        run(ts1_hbm, out1_hbm)


_edge_sc = functools.partial(
    pl.kernel,
    out_type=[jax.ShapeDtypeStruct((NPAD, F2), jnp.float32),
              jax.ShapeDtypeStruct((NPAD, F2), jnp.float32)],
    mesh=_mesh,
    compiler_params=pltpu.CompilerParams(use_tc_tiling_on_sc=False),
    scratch_types=[
        pltpu.VMEM_SHARED((NPAD, F2), jnp.float32),
        pltpu.VMEM((NIB, BLK, CH), jnp.int32),
        pltpu.VMEM((NIB, BLK, CH), jnp.int32),
        pltpu.VMEM((R, CH, F2), jnp.float32),
        pltpu.VMEM((CH, F2), jnp.float32),
        pltpu.SemaphoreType.DMA((R,)),
        pltpu.SemaphoreType.DMA((R,)),
        pltpu.SemaphoreType.DMA((NIB,)),
    ],
)(_edge_body)


def _prep_kernel(x_ref, w_ref, dg0_ref, dg1_ref, dinv_ref, ts0_ref, ts1_ref):
    deg = dg0_ref[...] + dg1_ref[...] + 1.0
    dinv = lax.rsqrt(deg)
    dinv_ref[...] = dinv
    ts = jnp.dot(x_ref[...], w_ref[...],
                 preferred_element_type=jnp.float32) * dinv
    ts0_ref[...] = ts[:, :F2]
    ts1_ref[...] = ts[:, F2:]


def _prep(xp, w1p, dg0, dg1):
    return pl.pallas_call(
        _prep_kernel,
        grid=(NPAD // BM,),
        in_specs=[pl.BlockSpec((BM, 16), lambda i: (i, 0)),
                  pl.BlockSpec((16, H), lambda i: (0, 0)),
                  pl.BlockSpec((BM, 1), lambda i: (i, 0)),
                  pl.BlockSpec((BM, 1), lambda i: (i, 0))],
        out_specs=[pl.BlockSpec((BM, 1), lambda i: (i, 0)),
                   pl.BlockSpec((BM, F2), lambda i: (i, 0)),
                   pl.BlockSpec((BM, F2), lambda i: (i, 0))],
        out_shape=[jax.ShapeDtypeStruct((NPAD, 1), jnp.float32),
                   jax.ShapeDtypeStruct((NPAD, F2), jnp.float32),
                   jax.ShapeDtypeStruct((NPAD, F2), jnp.float32)],
    )(xp, w1p, dg0, dg1)


def _combine_next_kernel(s0_ref, s1_ref, t0_ref, t1_ref, dinv_ref, b_ref,
                         w_ref, o0_ref, o1_ref):
    dinv = dinv_ref[...]
    h0 = jnp.maximum(dinv * (s0_ref[...] + t0_ref[...])
                     + b_ref[...][None, :F2], 0.0)
    h1 = jnp.maximum(dinv * (s1_ref[...] + t1_ref[...])
                     + b_ref[...][None, F2:], 0.0)
    h = jnp.concatenate([h0, h1], axis=1)
    ts = jnp.dot(h, w_ref[...], preferred_element_type=jnp.float32) * dinv
    o0_ref[...] = ts[:, :F2]
    o1_ref[...] = ts[:, F2:]


def _combine_next(s0, s1, t0, t1, dinv, b, w):
    return pl.pallas_call(
        _combine_next_kernel,
        grid=(NPAD // BM,),
        in_specs=[pl.BlockSpec((BM, F2), lambda i: (i, 0)),
                  pl.BlockSpec((BM, F2), lambda i: (i, 0)),
                  pl.BlockSpec((BM, F2), lambda i: (i, 0)),
                  pl.BlockSpec((BM, F2), lambda i: (i, 0)),
                  pl.BlockSpec((BM, 1), lambda i: (i, 0)),
                  pl.BlockSpec((H,), lambda i: (0,)),
                  pl.BlockSpec((H, H), lambda i: (0, 0))],
        out_specs=[pl.BlockSpec((BM, F2), lambda i: (i, 0)),
                   pl.BlockSpec((BM, F2), lambda i: (i, 0))],
        out_shape=[jax.ShapeDtypeStruct((NPAD, F2), jnp.float32),
                   jax.ShapeDtypeStruct((NPAD, F2), jnp.float32)],
    )(s0, s1, t0, t1, dinv, b, w)


def _combine_last_kernel(s0_ref, s1_ref, t0_ref, t1_ref, dinv_ref, b_ref,
                         h_ref):
    dinv = dinv_ref[...]
    h0 = jnp.maximum(dinv * (s0_ref[...] + t0_ref[...])
                     + b_ref[...][None, :F2], 0.0)
    h1 = jnp.maximum(dinv * (s1_ref[...] + t1_ref[...])
                     + b_ref[...][None, F2:], 0.0)
    h_ref[...] = jnp.concatenate([h0, h1], axis=1)


def _combine_last(s0, s1, t0, t1, dinv, b):
    return pl.pallas_call(
        _combine_last_kernel,
        grid=(NPAD // BM,),
        in_specs=[pl.BlockSpec((BM, F2), lambda i: (i, 0)),
                  pl.BlockSpec((BM, F2), lambda i: (i, 0)),
                  pl.BlockSpec((BM, F2), lambda i: (i, 0)),
                  pl.BlockSpec((BM, F2), lambda i: (i, 0)),
                  pl.BlockSpec((BM, 1), lambda i: (i, 0)),
                  pl.BlockSpec((H,), lambda i: (0,))],
        out_specs=pl.BlockSpec((BM, H), lambda i: (i, 0)),
        out_shape=jax.ShapeDtypeStruct((NPAD, H), jnp.float32),
    )(s0, s1, t0, t1, dinv, b)


def _meanpool_kernel(h_ref, batch_ref, sum_ref, cnt_ref):
    i = pl.program_id(0)

    @pl.when(i == 0)
    def _init():
        sum_ref[...] = jnp.zeros_like(sum_ref)
        cnt_ref[...] = jnp.zeros_like(cnt_ref)

    b = batch_ref[...]
    onehot = (b == jax.lax.broadcasted_iota(jnp.int32, (1, G), 1)
              ).astype(jnp.float32)
    sum_ref[...] += jnp.dot(onehot.T, h_ref[...],
                            preferred_element_type=jnp.float32)
    cnt_ref[...] += jnp.sum(onehot, axis=0)


def _meanpool(h, batch2d):
    return pl.pallas_call(
        _meanpool_kernel,
        grid=(NPAD // BM,),
        in_specs=[pl.BlockSpec((BM, H), lambda i: (i, 0)),
                  pl.BlockSpec((BM, 1), lambda i: (i, 0))],
        out_specs=[pl.BlockSpec((G, H), lambda i: (0, 0)),
                   pl.BlockSpec((G,), lambda i: (0,))],
        out_shape=[jax.ShapeDtypeStruct((G, H), jnp.float32),
                   jax.ShapeDtypeStruct((G,), jnp.float32)],
    )(h, batch2d)


def _head_kernel(sum_ref, cnt_ref, mx_ref, wp1_ref, bp1_ref, wp2_ref,
                 bp2_ref, o_ref):
    cnt = jnp.maximum(cnt_ref[...], 1.0)
    mean = sum_ref[...] / cnt[:, None]
    mx = mx_ref[...]
    mx = jnp.where(jnp.isfinite(mx), mx, 0.0)
    g = jnp.concatenate([mean, mx], axis=1)
    z = jnp.maximum(jnp.dot(g, wp1_ref[...],
                            preferred_element_type=jnp.float32)
                    + bp1_ref[...][None, :], 0.0)
    o_ref[...] = (jnp.dot(z, wp2_ref[...], preferred_element_type=jnp.float32)
                  + bp2_ref[...][None, :])


def _head(sums, cnt, mx, wp1, bp1, wp2, bp2):
    return pl.pallas_call(
        _head_kernel,
        out_shape=jax.ShapeDtypeStruct((G, 1), jnp.float32),
    )(sums, cnt, mx, wp1, bp1, wp2, bp2)


def kernel(x, edge_index, edge_attr, batch, W1, b1, W2, b2, W3, b3,
           Wp1, bp1, Wp2, bp2):
    del edge_attr
    src = jnp.concatenate([edge_index[0].astype(jnp.int32),
                           jnp.zeros((EPAD - E,), jnp.int32)])
    dst = jnp.concatenate([edge_index[1].astype(jnp.int32),
                           jnp.full((EPAD - E,), N, jnp.int32)])
    batch32 = batch.astype(jnp.int32)
    batch_p = jnp.concatenate([batch32, jnp.full((NPAD - N,), G, jnp.int32)])

    xp = jnp.pad(x, ((0, NPAD - N), (0, 16 - x.shape[1])))
    w1p = jnp.pad(W1, ((0, 16 - W1.shape[0]), (0, 0)))

    src2d = src.reshape(EPAD // CH, CH)
    dst2d = dst.reshape(EPAD // CH, CH)

    dg0, dg1 = _deg_sc(dst2d)
    dinv, t0, t1 = _prep(xp, w1p, dg0[:, None], dg1[:, None])

    s0, s1 = _edge_sc(src2d, dst2d, t0, t1)
    t0, t1 = _combine_next(s0, s1, t0, t1, dinv, b1, W2)
    s0, s1 = _edge_sc(src2d, dst2d, t0, t1)
    t0, t1 = _combine_next(s0, s1, t0, t1, dinv, b2, W3)
    s0, s1 = _edge_sc(src2d, dst2d, t0, t1)
    h3 = _combine_last(s0, s1, t0, t1, dinv, b3)

    sums, cnt = _meanpool(h3, batch_p[:, None])
    mx = jax.ops.segment_max(h3[:N], batch32, num_segments=G)
    return _head(sums, cnt, mx, Wp1, bp1, Wp2, bp2)
